# baseline (speedup 1.0000x reference)
import functools

import jax
import jax.numpy as jnp
import numpy as np
from jax import lax
from jax.experimental import pallas as pl
from jax.experimental.pallas import tpu as pltpu, tpu_sc as plsc

_B = 64
_N = 4096
_D = 1024
_DE = 128
_TEMP = 1.0
_MIN_TEMPERATURE = 1e-05
_INV_TEMP = 1.0 / max(float(_TEMP), _MIN_TEMPERATURE)

_NS = 2048
_NTC = _N - _NS
_G = 8
_CH = 256
_NW = 32
_GPW = _B // _NW


def _proj_body(ns_ref, wp_ref, wsp_ref, bs_ref, sp_ref, stop_ref):
    sp_ref[...] = jnp.dot(ns_ref[...], wp_ref[...],
                          preferred_element_type=jnp.float32)
    stop_ref[...] = jnp.dot(ns_ref[...], wsp_ref[...],
                            preferred_element_type=jnp.float32) + bs_ref[0, 0]


def _proj(node_states, W_proj, W_stop_pad, b_stop_2d):
    return pl.pallas_call(
        _proj_body,
        in_specs=[
            pl.BlockSpec((_B, _D), lambda: (0, 0)),
            pl.BlockSpec((_D, _DE), lambda: (0, 0)),
            pl.BlockSpec((_D, 128), lambda: (0, 0)),
            pl.BlockSpec(memory_space=pltpu.SMEM),
        ],
        out_specs=[
            pl.BlockSpec((_B, _DE), lambda: (0, 0)),
            pl.BlockSpec((_B, 128), lambda: (0, 0)),
        ],
        out_shape=[
            jax.ShapeDtypeStruct((_B, _DE), jnp.float32),
            jax.ShapeDtypeStruct((_B, 128), jnp.float32),
        ],
    )(node_states, W_proj, W_stop_pad, b_stop_2d)


_sc_mesh = plsc.VectorSubcoreMesh(core_axis_name="c", subcore_axis_name="s")


@functools.partial(
    pl.kernel,
    out_type=jax.ShapeDtypeStruct((_B, _NS), jnp.float32),
    mesh=_sc_mesh,
    scratch_types=[
        pltpu.VMEM((_CH, _DE), jnp.float32),
        pltpu.VMEM((_CH, _DE), jnp.float32),
        pltpu.VMEM((_DE,), jnp.float32),
        pltpu.VMEM((_NS,), jnp.float32),
        pltpu.VMEM((16, 16), jnp.float32),
        pltpu.SemaphoreType.DMA,
        pltpu.SemaphoreType.DMA,
    ],
    compiler_params=pltpu.CompilerParams(needs_layout_passes=False),
)
def _sc_score(ef_hbm, sp_hbm, out_hbm, buf0, buf1, spv, logit, tbuf,
              sem0, sem1):
    wid = lax.axis_index("s") * 2 + lax.axis_index("c")
    lane = lax.iota(jnp.int32, 16)
    nchunks = _NS // _CH

    def _bf16_round(v):
        u = plsc.bitcast(v, jnp.uint32)
        tie = (u >> jnp.uint32(16)) & jnp.uint32(1)
        r = (u + jnp.uint32(0x7FFF) + tie) & jnp.uint32(0xFFFF0000)
        return plsc.bitcast(r, jnp.float32)

    def _bf16_round_pair(a, b):
        return _bf16_round(a), _bf16_round(b)

    def do_graph(gi, carry):
        b = wid * _GPW + gi
        pltpu.sync_copy(sp_hbm.at[b], spv)
        spk = []
        for k in range(4):
            a, bb = _bf16_round_pair(spv[pl.ds(32 * k, 16)],
                                     spv[pl.ds(32 * k + 16, 16)])
            spk += [a, bb]

        bufs = (buf0, buf1)
        sems = (sem0, sem1)

        def chunk_cp(c, buf, sem):
            return pltpu.make_async_copy(
                ef_hbm.at[b, pl.ds(_NTC + c * _CH, _CH)], buf, sem)

        chunk_cp(0, bufs[0], sems[0]).start()
        for c in range(nchunks):
            cur, csem = bufs[c % 2], sems[c % 2]
            if c + 1 < nchunks:
                chunk_cp(c + 1, bufs[(c + 1) % 2], sems[(c + 1) % 2]).start()
            chunk_cp(c, cur, csem).wait()

            def grp(g, _, cur=cur, c=c):
                base = g * 16
                for j in range(16):
                    e = base + j
                    acc = None
                    for k in range(4):
                        v0, v1 = _bf16_round_pair(cur[e, pl.ds(32 * k, 16)],
                                                  cur[e, pl.ds(32 * k + 16, 16)])
                        t = spk[2 * k] * v0 + spk[2 * k + 1] * v1
                        acc = t if acc is None else acc + t
                    tbuf[j, :] = acc
                tot = plsc.load_gather(tbuf, [lane, jnp.zeros((16,), jnp.int32)])
                for col in range(1, 16):
                    tot = tot + plsc.load_gather(
                        tbuf, [lane, jnp.full((16,), col, jnp.int32)])
                logit[pl.ds(c * _CH + base, 16)] = tot
                return 0

            lax.fori_loop(0, _CH // 16, grp, 0)
        pltpu.sync_copy(logit, out_hbm.at[b])
        return 0

    lax.fori_loop(0, _GPW, do_graph, 0)


def _tc_edge_body(sp_ref, ef_ref, ge_ref, st_ref, x_scr):
    step = pl.program_id(0)
    base = step * _G
    for g in range(_G):
        sp_g = sp_ref[pl.ds(base + g, 1), :]
        x_scr[pl.ds(g, 1), :] = jax.lax.dot_general(
            sp_g, ef_ref[g], (((1,), (1,)), ((), ())),
            preferred_element_type=jnp.float32)

    x = x_scr[...] * _INV_TEMP
    m = jnp.max(x, axis=1, keepdims=True)
    s = jnp.sum(jnp.exp(x - m), axis=1, keepdims=True)
    pert = x + ge_ref[...]
    pm = jnp.max(pert, axis=1, keepdims=True)
    cols = jax.lax.broadcasted_iota(jnp.int32, (_G, _NTC), 1)
    pidx = jnp.min(jnp.where(pert == pm, cols, _NTC), axis=1, keepdims=True)
    plg = jnp.max(jnp.where(cols == pidx, x, -jnp.inf), axis=1, keepdims=True)
    st_ref[...] = jnp.concatenate(
        [m, s, pm, plg, pidx.astype(jnp.float32),
         jnp.zeros((_G, 123), jnp.float32)], axis=1)


def _tc_edges(sp, edge_feats, g_edges_tc):
    return pl.pallas_call(
        _tc_edge_body,
        grid=(_B // _G,),
        in_specs=[
            pl.BlockSpec((_B, _DE), lambda s: (0, 0)),
            pl.BlockSpec((_G, _NTC, _DE), lambda s: (s, 0, 0)),
            pl.BlockSpec((_G, _NTC), lambda s: (s, 0)),
        ],
        out_specs=pl.BlockSpec((_G, 128), lambda s: (s, 0)),
        out_shape=jax.ShapeDtypeStruct((_B, 128), jnp.float32),
        scratch_shapes=[pltpu.VMEM((_G, _NTC), jnp.float32)],
    )(sp, edge_feats, g_edges_tc)


def _merge_body(xs_ref, st_ref, stop_ref, gs_ref, gstop_ref,
                lpf_ref, act_ref):
    xs = xs_ref[...] * _INV_TEMP
    m_tc = st_ref[:, pl.ds(0, 1)]
    s_tc = st_ref[:, pl.ds(1, 1)]
    pm_tc = st_ref[:, pl.ds(2, 1)]
    plg_tc = st_ref[:, pl.ds(3, 1)]
    pidx_tc = st_ref[:, pl.ds(4, 1)].astype(jnp.int32)
    stop_l = stop_ref[:, pl.ds(0, 1)] * _INV_TEMP

    m_sc = jnp.max(xs, axis=1, keepdims=True)
    s_sc = jnp.sum(jnp.exp(xs - m_sc), axis=1, keepdims=True)
    pert = xs + gs_ref[...]
    pm_sc = jnp.max(pert, axis=1, keepdims=True)
    cols = jax.lax.broadcasted_iota(jnp.int32, (_B, _NS), 1)
    pidx_sc = jnp.min(jnp.where(pert == pm_sc, cols, _NS),
                      axis=1, keepdims=True)
    plg_sc = jnp.max(jnp.where(cols == pidx_sc, xs, -jnp.inf),
                     axis=1, keepdims=True)

    m = jnp.maximum(jnp.maximum(m_tc, m_sc), stop_l)
    lse = m + jnp.log(s_tc * jnp.exp(m_tc - m) + s_sc * jnp.exp(m_sc - m)
                      + jnp.exp(stop_l - m))

    tc_wins = pm_tc >= pm_sc
    pm_e = jnp.maximum(pm_tc, pm_sc)
    eidx = jnp.where(tc_wins, pidx_tc, _NTC + pidx_sc)
    elg = jnp.where(tc_wins, plg_tc, plg_sc)
    pert_stop = stop_l + gstop_ref[:, pl.ds(0, 1)]
    take_stop = pert_stop > pm_e
    action = jnp.where(take_stop, _N, eidx)
    log_pf = jnp.where(take_stop, stop_l, elg) - lse

    lpf_ref[...] = jnp.broadcast_to(log_pf, (_B, 128))
    act_ref[...] = jnp.broadcast_to(action, (_B, 128))


def _merge(x_sc, stats, stop, g_edges_sc, g_stop):
    return pl.pallas_call(
        _merge_body,
        in_specs=[
            pl.BlockSpec((_B, _NS), lambda: (0, 0)),
            pl.BlockSpec((_B, 128), lambda: (0, 0)),
            pl.BlockSpec((_B, 128), lambda: (0, 0)),
            pl.BlockSpec((_B, _NS), lambda: (0, 0)),
            pl.BlockSpec((_B, 128), lambda: (0, 0)),
        ],
        out_specs=[
            pl.BlockSpec((_B, 128), lambda: (0, 0)),
            pl.BlockSpec((_B, 128), lambda: (0, 0)),
        ],
        out_shape=[
            jax.ShapeDtypeStruct((_B, 128), jnp.float32),
            jax.ShapeDtypeStruct((_B, 128), jnp.int32),
        ],
    )(x_sc, stats, stop, g_edges_sc, g_stop)


@jax.jit
def _run(node_states, edge_feats, W_proj, W_stop_pad, b_stop_2d,
         g_edges_tc, g_edges_sc, g_stop):
    sp, stop = _proj(node_states, W_proj, W_stop_pad, b_stop_2d)
    x_sc = _sc_score(edge_feats, sp)
    stats = _tc_edges(sp, edge_feats, g_edges_tc)
    lpf, act = _merge(x_sc, stats, stop, g_edges_sc, g_stop)
    return lpf[:, 0], act[:, 0]


def kernel(node_states, edge_feats, W_proj, W_stop, b_stop):
    u = jax.random.uniform(jax.random.key(1), (_B, _N + 1),
                           dtype=jnp.float32, minval=1e-9, maxval=1.0)
    gumbel = -jnp.log(-jnp.log(u))
    g_edges_tc = gumbel[:, :_NTC]
    g_edges_sc = gumbel[:, _NTC:_N]
    g_stop = jnp.broadcast_to(gumbel[:, _N:], (_B, 128))
    W_stop_pad = jnp.pad(W_stop, ((0, 0), (0, 127)))
    b_stop_2d = b_stop.reshape(1, 1)
    return _run(node_states, edge_feats, W_proj, W_stop_pad, b_stop_2d,
                g_edges_tc, g_edges_sc, g_stop)


# speedup vs baseline: 1.9136x; 1.9136x over previous
import functools

import jax
import jax.numpy as jnp
import numpy as np
from jax import lax
from jax.experimental import pallas as pl
from jax.experimental.pallas import tpu as pltpu, tpu_sc as plsc

_B = 64
_N = 4096
_D = 1024
_DE = 128
_TEMP = 1.0
_MIN_TEMPERATURE = 1e-05
_INV_TEMP = 1.0 / max(float(_TEMP), _MIN_TEMPERATURE)

_NS = 768
_NTC = _N - _NS
_G = 8
_CH = 256
_NW = 32
_GPW = _B // _NW


def _proj_body(ns_ref, wp_ref, wsp_ref, bs_ref, sp_ref, stop_ref):
    sp_ref[...] = jnp.dot(ns_ref[...], wp_ref[...],
                          preferred_element_type=jnp.float32)
    stop_ref[...] = jnp.dot(ns_ref[...], wsp_ref[...],
                            preferred_element_type=jnp.float32) + bs_ref[0, 0]


def _proj(node_states, W_proj, W_stop_pad, b_stop_2d):
    return pl.pallas_call(
        _proj_body,
        in_specs=[
            pl.BlockSpec((_B, _D), lambda: (0, 0)),
            pl.BlockSpec((_D, _DE), lambda: (0, 0)),
            pl.BlockSpec((_D, 128), lambda: (0, 0)),
            pl.BlockSpec(memory_space=pltpu.SMEM),
        ],
        out_specs=[
            pl.BlockSpec((_B, _DE), lambda: (0, 0)),
            pl.BlockSpec((_B, 128), lambda: (0, 0)),
        ],
        out_shape=[
            jax.ShapeDtypeStruct((_B, _DE), jnp.float32),
            jax.ShapeDtypeStruct((_B, 128), jnp.float32),
        ],
    )(node_states, W_proj, W_stop_pad, b_stop_2d)


_sc_mesh = plsc.VectorSubcoreMesh(core_axis_name="c", subcore_axis_name="s")


@functools.partial(
    pl.kernel,
    out_type=jax.ShapeDtypeStruct((_B, _NS), jnp.float32),
    mesh=_sc_mesh,
    scratch_types=[
        pltpu.VMEM((_CH, _DE), jnp.float32),
        pltpu.VMEM((_CH, _DE), jnp.float32),
        pltpu.VMEM((_DE,), jnp.float32),
        pltpu.VMEM((_NS,), jnp.float32),
        pltpu.VMEM((16, 16), jnp.float32),
        pltpu.SemaphoreType.DMA,
        pltpu.SemaphoreType.DMA,
    ],
    compiler_params=pltpu.CompilerParams(needs_layout_passes=False),
)
def _sc_score(ef_hbm, sp_hbm, out_hbm, buf0, buf1, spv, logit, tbuf,
              sem0, sem1):
    wid = lax.axis_index("s") * 2 + lax.axis_index("c")
    lane = lax.iota(jnp.int32, 16)
    nchunks = _NS // _CH

    def _bf16_round(v):
        u = plsc.bitcast(v, jnp.uint32)
        tie = (u >> jnp.uint32(16)) & jnp.uint32(1)
        r = (u + jnp.uint32(0x7FFF) + tie) & jnp.uint32(0xFFFF0000)
        return plsc.bitcast(r, jnp.float32)

    def _bf16_round_pair(a, b):
        return _bf16_round(a), _bf16_round(b)

    def do_graph(gi, carry):
        b = wid * _GPW + gi
        pltpu.sync_copy(sp_hbm.at[b], spv)
        spk = []
        for k in range(4):
            a, bb = _bf16_round_pair(spv[pl.ds(32 * k, 16)],
                                     spv[pl.ds(32 * k + 16, 16)])
            spk += [a, bb]

        bufs = (buf0, buf1)
        sems = (sem0, sem1)

        def chunk_cp(c, buf, sem):
            return pltpu.make_async_copy(
                ef_hbm.at[b, pl.ds(_NTC + c * _CH, _CH)], buf, sem)

        chunk_cp(0, bufs[0], sems[0]).start()
        for c in range(nchunks):
            cur, csem = bufs[c % 2], sems[c % 2]
            if c + 1 < nchunks:
                chunk_cp(c + 1, bufs[(c + 1) % 2], sems[(c + 1) % 2]).start()
            chunk_cp(c, cur, csem).wait()

            def grp(g, _, cur=cur, c=c):
                base = g * 16
                for j in range(16):
                    e = base + j
                    acc = None
                    for k in range(4):
                        v0, v1 = _bf16_round_pair(cur[e, pl.ds(32 * k, 16)],
                                                  cur[e, pl.ds(32 * k + 16, 16)])
                        t = spk[2 * k] * v0 + spk[2 * k + 1] * v1
                        acc = t if acc is None else acc + t
                    tbuf[j, :] = acc
                tot = plsc.load_gather(tbuf, [lane, jnp.zeros((16,), jnp.int32)])
                for col in range(1, 16):
                    tot = tot + plsc.load_gather(
                        tbuf, [lane, jnp.full((16,), col, jnp.int32)])
                logit[pl.ds(c * _CH + base, 16)] = tot
                return 0

            lax.fori_loop(0, _CH // 16, grp, 0)
        pltpu.sync_copy(logit, out_hbm.at[b])
        return 0

    lax.fori_loop(0, _GPW, do_graph, 0)


def _tc_edge_body(sp_ref, ef_ref, ge_ref, st_ref, x_scr):
    step = pl.program_id(0)
    base = step * _G
    for g in range(_G):
        sp_g = sp_ref[pl.ds(base + g, 1), :]
        x_scr[pl.ds(g, 1), :] = jax.lax.dot_general(
            sp_g, ef_ref[g], (((1,), (1,)), ((), ())),
            preferred_element_type=jnp.float32)

    x = x_scr[...] * _INV_TEMP
    m = jnp.max(x, axis=1, keepdims=True)
    s = jnp.sum(jnp.exp(x - m), axis=1, keepdims=True)
    pert = x + ge_ref[...]
    pm = jnp.max(pert, axis=1, keepdims=True)
    cols = jax.lax.broadcasted_iota(jnp.int32, (_G, _NTC), 1)
    pidx = jnp.min(jnp.where(pert == pm, cols, _NTC), axis=1, keepdims=True)
    plg = jnp.max(jnp.where(cols == pidx, x, -jnp.inf), axis=1, keepdims=True)
    st_ref[...] = jnp.concatenate(
        [m, s, pm, plg, pidx.astype(jnp.float32),
         jnp.zeros((_G, 123), jnp.float32)], axis=1)


def _tc_edges(sp, edge_feats, g_edges_tc):
    return pl.pallas_call(
        _tc_edge_body,
        grid=(_B // _G,),
        in_specs=[
            pl.BlockSpec((_B, _DE), lambda s: (0, 0)),
            pl.BlockSpec((_G, _NTC, _DE), lambda s: (s, 0, 0)),
            pl.BlockSpec((_G, _NTC), lambda s: (s, 0)),
        ],
        out_specs=pl.BlockSpec((_G, 128), lambda s: (s, 0)),
        out_shape=jax.ShapeDtypeStruct((_B, 128), jnp.float32),
        scratch_shapes=[pltpu.VMEM((_G, _NTC), jnp.float32)],
    )(sp, edge_feats, g_edges_tc)


def _merge_body(xs_ref, st_ref, stop_ref, gs_ref, gstop_ref,
                lpf_ref, act_ref):
    xs = xs_ref[...] * _INV_TEMP
    m_tc = st_ref[:, pl.ds(0, 1)]
    s_tc = st_ref[:, pl.ds(1, 1)]
    pm_tc = st_ref[:, pl.ds(2, 1)]
    plg_tc = st_ref[:, pl.ds(3, 1)]
    pidx_tc = st_ref[:, pl.ds(4, 1)].astype(jnp.int32)
    stop_l = stop_ref[:, pl.ds(0, 1)] * _INV_TEMP

    m_sc = jnp.max(xs, axis=1, keepdims=True)
    s_sc = jnp.sum(jnp.exp(xs - m_sc), axis=1, keepdims=True)
    pert = xs + gs_ref[...]
    pm_sc = jnp.max(pert, axis=1, keepdims=True)
    cols = jax.lax.broadcasted_iota(jnp.int32, (_B, _NS), 1)
    pidx_sc = jnp.min(jnp.where(pert == pm_sc, cols, _NS),
                      axis=1, keepdims=True)
    plg_sc = jnp.max(jnp.where(cols == pidx_sc, xs, -jnp.inf),
                     axis=1, keepdims=True)

    m = jnp.maximum(jnp.maximum(m_tc, m_sc), stop_l)
    lse = m + jnp.log(s_tc * jnp.exp(m_tc - m) + s_sc * jnp.exp(m_sc - m)
                      + jnp.exp(stop_l - m))

    tc_wins = pm_tc >= pm_sc
    pm_e = jnp.maximum(pm_tc, pm_sc)
    eidx = jnp.where(tc_wins, pidx_tc, _NTC + pidx_sc)
    elg = jnp.where(tc_wins, plg_tc, plg_sc)
    pert_stop = stop_l + gstop_ref[:, pl.ds(0, 1)]
    take_stop = pert_stop > pm_e
    action = jnp.where(take_stop, _N, eidx)
    log_pf = jnp.where(take_stop, stop_l, elg) - lse

    lpf_ref[...] = jnp.broadcast_to(log_pf, (_B, 128))
    act_ref[...] = jnp.broadcast_to(action, (_B, 128))


def _merge(x_sc, stats, stop, g_edges_sc, g_stop):
    return pl.pallas_call(
        _merge_body,
        in_specs=[
            pl.BlockSpec((_B, _NS), lambda: (0, 0)),
            pl.BlockSpec((_B, 128), lambda: (0, 0)),
            pl.BlockSpec((_B, 128), lambda: (0, 0)),
            pl.BlockSpec((_B, _NS), lambda: (0, 0)),
            pl.BlockSpec((_B, 128), lambda: (0, 0)),
        ],
        out_specs=[
            pl.BlockSpec((_B, 128), lambda: (0, 0)),
            pl.BlockSpec((_B, 128), lambda: (0, 0)),
        ],
        out_shape=[
            jax.ShapeDtypeStruct((_B, 128), jnp.float32),
            jax.ShapeDtypeStruct((_B, 128), jnp.int32),
        ],
    )(x_sc, stats, stop, g_edges_sc, g_stop)


@jax.jit
def _run(node_states, edge_feats, W_proj, W_stop_pad, b_stop_2d,
         g_edges_tc, g_edges_sc, g_stop):
    sp, stop = _proj(node_states, W_proj, W_stop_pad, b_stop_2d)
    x_sc = _sc_score(edge_feats, sp)
    stats = _tc_edges(sp, edge_feats, g_edges_tc)
    lpf, act = _merge(x_sc, stats, stop, g_edges_sc, g_stop)
    return lpf[:, 0], act[:, 0]


def kernel(node_states, edge_feats, W_proj, W_stop, b_stop):
    u = jax.random.uniform(jax.random.key(1), (_B, _N + 1),
                           dtype=jnp.float32, minval=1e-9, maxval=1.0)
    gumbel = -jnp.log(-jnp.log(u))
    g_edges_tc = gumbel[:, :_NTC]
    g_edges_sc = gumbel[:, _NTC:_N]
    g_stop = jnp.broadcast_to(gumbel[:, _N:], (_B, 128))
    W_stop_pad = jnp.pad(W_stop, ((0, 0), (0, 127)))
    b_stop_2d = b_stop.reshape(1, 1)
    return _run(node_states, edge_feats, W_proj, W_stop_pad, b_stop_2d,
                g_edges_tc, g_edges_sc, g_stop)


# speedup vs baseline: 2.1068x; 1.1010x over previous
import functools

import jax
import jax.numpy as jnp
import numpy as np
from jax import lax
from jax.experimental import pallas as pl
from jax.experimental.pallas import tpu as pltpu, tpu_sc as plsc

_B = 64
_N = 4096
_D = 1024
_DE = 128
_TEMP = 1.0
_MIN_TEMPERATURE = 1e-05
_INV_TEMP = 1.0 / max(float(_TEMP), _MIN_TEMPERATURE)

_NS = 256
_NTC = _N - _NS
_G = 8
_CH = 256
_NW = 32
_GPW = _B // _NW


def _proj_body(ns_ref, wp_ref, wsp_ref, bs_ref, sp_ref, stop_ref):
    sp_ref[...] = jnp.dot(ns_ref[...], wp_ref[...],
                          preferred_element_type=jnp.float32)
    stop_ref[...] = jnp.dot(ns_ref[...], wsp_ref[...],
                            preferred_element_type=jnp.float32) + bs_ref[0, 0]


def _proj(node_states, W_proj, W_stop_pad, b_stop_2d):
    return pl.pallas_call(
        _proj_body,
        in_specs=[
            pl.BlockSpec((_B, _D), lambda: (0, 0)),
            pl.BlockSpec((_D, _DE), lambda: (0, 0)),
            pl.BlockSpec((_D, 128), lambda: (0, 0)),
            pl.BlockSpec(memory_space=pltpu.SMEM),
        ],
        out_specs=[
            pl.BlockSpec((_B, _DE), lambda: (0, 0)),
            pl.BlockSpec((_B, 128), lambda: (0, 0)),
        ],
        out_shape=[
            jax.ShapeDtypeStruct((_B, _DE), jnp.float32),
            jax.ShapeDtypeStruct((_B, 128), jnp.float32),
        ],
    )(node_states, W_proj, W_stop_pad, b_stop_2d)


_sc_mesh = plsc.VectorSubcoreMesh(core_axis_name="c", subcore_axis_name="s")


@functools.partial(
    pl.kernel,
    out_type=jax.ShapeDtypeStruct((_B, _NS), jnp.float32),
    mesh=_sc_mesh,
    scratch_types=[
        pltpu.VMEM((_CH, _DE), jnp.float32),
        pltpu.VMEM((_CH, _DE), jnp.float32),
        pltpu.VMEM((_DE,), jnp.float32),
        pltpu.VMEM((_NS,), jnp.float32),
        pltpu.VMEM((16, 16), jnp.float32),
        pltpu.SemaphoreType.DMA,
        pltpu.SemaphoreType.DMA,
    ],
    compiler_params=pltpu.CompilerParams(needs_layout_passes=False),
)
def _sc_score(ef_hbm, sp_hbm, out_hbm, buf0, buf1, spv, logit, tbuf,
              sem0, sem1):
    wid = lax.axis_index("s") * 2 + lax.axis_index("c")
    lane = lax.iota(jnp.int32, 16)
    nchunks = _NS // _CH

    def _bf16_round(v):
        u = plsc.bitcast(v, jnp.uint32)
        tie = (u >> jnp.uint32(16)) & jnp.uint32(1)
        r = (u + jnp.uint32(0x7FFF) + tie) & jnp.uint32(0xFFFF0000)
        return plsc.bitcast(r, jnp.float32)

    def _bf16_round_pair(a, b):
        return _bf16_round(a), _bf16_round(b)

    def do_graph(gi, carry):
        b = wid * _GPW + gi
        pltpu.sync_copy(sp_hbm.at[b], spv)
        spk = []
        for k in range(4):
            a, bb = _bf16_round_pair(spv[pl.ds(32 * k, 16)],
                                     spv[pl.ds(32 * k + 16, 16)])
            spk += [a, bb]

        bufs = (buf0, buf1)
        sems = (sem0, sem1)

        def chunk_cp(c, buf, sem):
            return pltpu.make_async_copy(
                ef_hbm.at[b, pl.ds(_NTC + c * _CH, _CH)], buf, sem)

        chunk_cp(0, bufs[0], sems[0]).start()
        for c in range(nchunks):
            cur, csem = bufs[c % 2], sems[c % 2]
            if c + 1 < nchunks:
                chunk_cp(c + 1, bufs[(c + 1) % 2], sems[(c + 1) % 2]).start()
            chunk_cp(c, cur, csem).wait()

            def grp(g, _, cur=cur, c=c):
                base = g * 16
                for j in range(16):
                    e = base + j
                    acc = None
                    for k in range(4):
                        v0, v1 = _bf16_round_pair(cur[e, pl.ds(32 * k, 16)],
                                                  cur[e, pl.ds(32 * k + 16, 16)])
                        t = spk[2 * k] * v0 + spk[2 * k + 1] * v1
                        acc = t if acc is None else acc + t
                    tbuf[j, :] = acc
                tot = plsc.load_gather(tbuf, [lane, jnp.zeros((16,), jnp.int32)])
                for col in range(1, 16):
                    tot = tot + plsc.load_gather(
                        tbuf, [lane, jnp.full((16,), col, jnp.int32)])
                logit[pl.ds(c * _CH + base, 16)] = tot
                return 0

            lax.fori_loop(0, _CH // 16, grp, 0)
        pltpu.sync_copy(logit, out_hbm.at[b])
        return 0

    lax.fori_loop(0, _GPW, do_graph, 0)


def _tc_edge_body(sp_ref, ef_ref, ge_ref, st_ref, x_scr):
    step = pl.program_id(0)
    base = step * _G
    for g in range(_G):
        sp_g = sp_ref[pl.ds(base + g, 1), :]
        x_scr[pl.ds(g, 1), :] = jax.lax.dot_general(
            sp_g, ef_ref[g], (((1,), (1,)), ((), ())),
            preferred_element_type=jnp.float32)

    x = x_scr[...] * _INV_TEMP
    m = jnp.max(x, axis=1, keepdims=True)
    s = jnp.sum(jnp.exp(x - m), axis=1, keepdims=True)
    pert = x + ge_ref[...]
    pm = jnp.max(pert, axis=1, keepdims=True)
    cols = jax.lax.broadcasted_iota(jnp.int32, (_G, _NTC), 1)
    pidx = jnp.min(jnp.where(pert == pm, cols, _NTC), axis=1, keepdims=True)
    plg = jnp.max(jnp.where(cols == pidx, x, -jnp.inf), axis=1, keepdims=True)
    st_ref[...] = jnp.concatenate(
        [m, s, pm, plg, pidx.astype(jnp.float32),
         jnp.zeros((_G, 123), jnp.float32)], axis=1)


def _tc_edges(sp, edge_feats, g_edges_tc):
    return pl.pallas_call(
        _tc_edge_body,
        grid=(_B // _G,),
        in_specs=[
            pl.BlockSpec((_B, _DE), lambda s: (0, 0)),
            pl.BlockSpec((_G, _NTC, _DE), lambda s: (s, 0, 0)),
            pl.BlockSpec((_G, _NTC), lambda s: (s, 0)),
        ],
        out_specs=pl.BlockSpec((_G, 128), lambda s: (s, 0)),
        out_shape=jax.ShapeDtypeStruct((_B, 128), jnp.float32),
        scratch_shapes=[pltpu.VMEM((_G, _NTC), jnp.float32)],
    )(sp, edge_feats, g_edges_tc)


def _merge_body(xs_ref, st_ref, stop_ref, gs_ref, gstop_ref,
                lpf_ref, act_ref):
    xs = xs_ref[...] * _INV_TEMP
    m_tc = st_ref[:, pl.ds(0, 1)]
    s_tc = st_ref[:, pl.ds(1, 1)]
    pm_tc = st_ref[:, pl.ds(2, 1)]
    plg_tc = st_ref[:, pl.ds(3, 1)]
    pidx_tc = st_ref[:, pl.ds(4, 1)].astype(jnp.int32)
    stop_l = stop_ref[:, pl.ds(0, 1)] * _INV_TEMP

    m_sc = jnp.max(xs, axis=1, keepdims=True)
    s_sc = jnp.sum(jnp.exp(xs - m_sc), axis=1, keepdims=True)
    pert = xs + gs_ref[...]
    pm_sc = jnp.max(pert, axis=1, keepdims=True)
    cols = jax.lax.broadcasted_iota(jnp.int32, (_B, _NS), 1)
    pidx_sc = jnp.min(jnp.where(pert == pm_sc, cols, _NS),
                      axis=1, keepdims=True)
    plg_sc = jnp.max(jnp.where(cols == pidx_sc, xs, -jnp.inf),
                     axis=1, keepdims=True)

    m = jnp.maximum(jnp.maximum(m_tc, m_sc), stop_l)
    lse = m + jnp.log(s_tc * jnp.exp(m_tc - m) + s_sc * jnp.exp(m_sc - m)
                      + jnp.exp(stop_l - m))

    tc_wins = pm_tc >= pm_sc
    pm_e = jnp.maximum(pm_tc, pm_sc)
    eidx = jnp.where(tc_wins, pidx_tc, _NTC + pidx_sc)
    elg = jnp.where(tc_wins, plg_tc, plg_sc)
    pert_stop = stop_l + gstop_ref[:, pl.ds(0, 1)]
    take_stop = pert_stop > pm_e
    action = jnp.where(take_stop, _N, eidx)
    log_pf = jnp.where(take_stop, stop_l, elg) - lse

    lpf_ref[...] = jnp.broadcast_to(log_pf, (_B, 128))
    act_ref[...] = jnp.broadcast_to(action, (_B, 128))


def _merge(x_sc, stats, stop, g_edges_sc, g_stop):
    return pl.pallas_call(
        _merge_body,
        in_specs=[
            pl.BlockSpec((_B, _NS), lambda: (0, 0)),
            pl.BlockSpec((_B, 128), lambda: (0, 0)),
            pl.BlockSpec((_B, 128), lambda: (0, 0)),
            pl.BlockSpec((_B, _NS), lambda: (0, 0)),
            pl.BlockSpec((_B, 128), lambda: (0, 0)),
        ],
        out_specs=[
            pl.BlockSpec((_B, 128), lambda: (0, 0)),
            pl.BlockSpec((_B, 128), lambda: (0, 0)),
        ],
        out_shape=[
            jax.ShapeDtypeStruct((_B, 128), jnp.float32),
            jax.ShapeDtypeStruct((_B, 128), jnp.int32),
        ],
    )(x_sc, stats, stop, g_edges_sc, g_stop)


@jax.jit
def _run(node_states, edge_feats, W_proj, W_stop_pad, b_stop_2d,
         g_edges_tc, g_edges_sc, g_stop):
    sp, stop = _proj(node_states, W_proj, W_stop_pad, b_stop_2d)
    x_sc = _sc_score(edge_feats, sp)
    stats = _tc_edges(sp, edge_feats, g_edges_tc)
    lpf, act = _merge(x_sc, stats, stop, g_edges_sc, g_stop)
    return lpf[:, 0], act[:, 0]


def kernel(node_states, edge_feats, W_proj, W_stop, b_stop):
    u = jax.random.uniform(jax.random.key(1), (_B, _N + 1),
                           dtype=jnp.float32, minval=1e-9, maxval=1.0)
    gumbel = -jnp.log(-jnp.log(u))
    g_edges_tc = gumbel[:, :_NTC]
    g_edges_sc = gumbel[:, _NTC:_N]
    g_stop = jnp.broadcast_to(gumbel[:, _N:], (_B, 128))
    W_stop_pad = jnp.pad(W_stop, ((0, 0), (0, 127)))
    b_stop_2d = b_stop.reshape(1, 1)
    return _run(node_states, edge_feats, W_proj, W_stop_pad, b_stop_2d,
                g_edges_tc, g_edges_sc, g_stop)
